# baseline (speedup 1.0000x reference)
import jax
import jax.numpy as jnp
import numpy as np
from jax.experimental import pallas as pl
from jax.experimental.pallas import tpu as pltpu

_PS = 32
_NB = 36
_BB = 512
_K = 512


def _gauss_row():
    half = _PS / 2.0
    sigma2 = 0.9 * half * half
    x = np.linspace(-half, half, _PS)
    xv, yv = np.meshgrid(x, x, indexing="xy")
    k = np.exp(-(xv ** 2 + yv ** 2) / sigma2)
    k = k / np.sum(k)
    k32 = k.astype(np.float32)
    return (np.float32(10.0) * k32).reshape(1, _PS * _PS)


def _body(x_ref, gk_ref, o_ref):
    x = x_ref[...]
    n = _PS * _PS

    lane = jax.lax.broadcasted_iota(jnp.int32, (1, n), 1)
    col = lane % _PS
    row = lane // _PS

    xl = jnp.where(col == 0, x, pltpu.roll(x, 1, axis=1))
    xr = jnp.where(col == _PS - 1, x, pltpu.roll(x, n - 1, axis=1))
    gx = 0.5 * (xl - xr)

    xu = jnp.where(row == 0, x, pltpu.roll(x, _PS, axis=1))
    xd = jnp.where(row == _PS - 1, x, pltpu.roll(x, n - _PS, axis=1))
    gy = 0.5 * (xu - xd)

    gk = gk_ref[...]
    mag = jnp.sqrt(gx * gx + gy * gy + 1e-10) * gk
    ori = jnp.arctan2(gy, gx)

    o_big = _NB * (ori + np.float32(np.pi)) / np.float32(2.0 * np.pi)
    bo0 = jnp.floor(o_big)
    w = (1.0 - (o_big - bo0)) * mag
    bo = jnp.where(bo0 >= _NB, np.float32(0.0), bo0)

    dist = jnp.abs(o_big - jnp.round(o_big))
    pot = jnp.sum(jnp.where(dist < np.float32(1.2e-5), mag, 0.0), axis=1,
                  keepdims=True)

    inv_n = np.float32(1.0 / n)
    cols = []
    for k in range(_NB):
        m = bo == np.float32(k)
        cols.append(jnp.sum(jnp.where(m, w, 0.0), axis=1, keepdims=True))
    hist = jnp.concatenate(cols, axis=1) * inv_n

    z = jnp.zeros((hist.shape[0], 1), jnp.float32)
    hl = jnp.concatenate([z, hist[:, :-1]], axis=1)
    hr = jnp.concatenate([hist[:, 1:], z], axis=1)
    sm = 0.33 * hl + 0.34 * hist + 0.33 * hr

    mx = jnp.max(sm, axis=1, keepdims=True)
    io = jax.lax.broadcasted_iota(jnp.int32, sm.shape, 1).astype(jnp.float32)
    eq = sm == mx
    idx = jnp.min(jnp.where(eq, io, np.float32(_NB)), axis=1, keepdims=True)
    ang = -(np.float32(2.0 * np.pi / _NB) * idx - np.float32(np.pi))

    sm2 = jnp.max(jnp.where(io == idx, np.float32(-1.0), sm), axis=1,
                  keepdims=True)
    score = (mx - sm2) - pot * np.float32(0.4 / n)

    packed = jnp.where(lane[:, :128] < 64, ang, score)
    o_ref[...] = packed


def _ref_formula(xb):
    b = xb.shape[0]
    gk = jnp.asarray(_gauss_row().reshape(_PS, _PS))
    xp = jnp.pad(xb, ((0, 0), (0, 0), (0, 0), (1, 1)), mode="edge")
    gx = 0.5 * (xp[..., :-2] - xp[..., 2:])
    yp = jnp.pad(xb, ((0, 0), (0, 0), (1, 1), (0, 0)), mode="edge")
    gy = 0.5 * (yp[:, :, :-2, :] - yp[:, :, 2:, :])
    mag = jnp.sqrt(gx * gx + gy * gy + 1e-10) * gk
    ori = jnp.arctan2(gy, gx)
    o_big = _NB * (ori + jnp.pi) / (2.0 * jnp.pi)
    bo0 = jnp.floor(o_big)
    wo1 = o_big - bo0
    bo0 = (bo0 % _NB).astype(jnp.int32)
    wo0 = (1.0 - wo1) * mag
    bins_flat = bo0.reshape(b, -1)
    w_flat = wo0.reshape(b, -1)
    hist = jax.vmap(
        lambda w_, b_: jax.ops.segment_sum(w_, b_, num_segments=_NB)
    )(w_flat, bins_flat) / float(_PS * _PS)
    hp = jnp.pad(hist, ((0, 0), (1, 1)))
    sm = 0.33 * hp[:, :-2] + 0.34 * hp[:, 1:-1] + 0.33 * hp[:, 2:]
    idx = jnp.argmax(sm, axis=1)
    return -(2.0 * jnp.pi * idx.astype(xb.dtype) / _NB - jnp.pi)


@jax.jit
def kernel(x):
    b = x.shape[0]
    x2 = x.reshape(b, _PS * _PS)
    grid = (b // _BB,)
    out = pl.pallas_call(
        _body,
        grid=grid,
        in_specs=[
            pl.BlockSpec((_BB, _PS * _PS), lambda i: (i, 0)),
            pl.BlockSpec((1, _PS * _PS), lambda i: (0, 0)),
        ],
        out_specs=pl.BlockSpec((_BB, 128), lambda i: (i, 0)),
        out_shape=jax.ShapeDtypeStruct((b, 128), jnp.float32),
        compiler_params=pltpu.CompilerParams(
            dimension_semantics=("parallel",),
        ),
    )(x2, jnp.asarray(_gauss_row()))
    ang = out[:, 0]
    score = out[:, 64]
    _, risky = jax.lax.top_k(-score, _K)
    xr = x2[risky].reshape(_K, 1, _PS, _PS)
    ang_exact = _ref_formula(xr)
    return ang.at[risky].set(ang_exact)
